# baseline (speedup 1.0000x reference)
import functools

import jax
import jax.numpy as jnp
from jax import lax
from jax.experimental import pallas as pl
from jax.experimental.pallas import tpu as pltpu
from jax.experimental.pallas import tpu_sc as plsc

N = 10000
D = 128
E = 320000

NC, NS, L = 2, 16, 16
NW = NC * NS
CHUNK = 128
EPT = 10240
NCHUNK = EPT // CHUNK
EPAD = EPT * NW
IDX_ROWS = EPAD // CHUNK
NPAD = 10240
RPT = NPAD // NS

_f32 = jnp.float32


def _deg_body(dst_hbm, deg0_hbm, deg1_hbm, deg_acc, dst_v, ones_v, zeros_v):
    c = lax.axis_index("c")
    s = lax.axis_index("s")
    w = c * NS + s

    def fill_zeros(i, carry):
        zeros_v[pl.ds(i * L, L)] = jnp.zeros((L,), _f32)
        return carry

    lax.fori_loop(0, RPT // L, fill_zeros, 0)

    def fill_ones(i, carry):
        ones_v[pl.ds(i * L, L)] = jnp.ones((L,), _f32)
        return carry

    lax.fori_loop(0, CHUNK // L, fill_ones, 0)

    pltpu.sync_copy(zeros_v, deg_acc.at[pl.ds(s * RPT, RPT)])
    pltpu.sync_copy(dst_hbm.at[pl.ds(w * NCHUNK, NCHUNK)], dst_v)
    plsc.subcore_barrier()

    def body(k, carry):
        pltpu.sync_copy(ones_v, deg_acc.at[dst_v.at[k]], add=True)
        return carry

    lax.fori_loop(0, NCHUNK, body, 0)
    plsc.subcore_barrier()

    rows = pl.ds(s * RPT, RPT)

    @pl.when(c == 0)
    def _():
        pltpu.sync_copy(deg_acc.at[rows], deg0_hbm.at[rows])

    @pl.when(c == 1)
    def _():
        pltpu.sync_copy(deg_acc.at[rows], deg1_hbm.at[rows])


@functools.lru_cache(maxsize=None)
def _deg_call():
    mesh = plsc.VectorSubcoreMesh(
        core_axis_name="c", subcore_axis_name="s", num_cores=NC, num_subcores=NS
    )
    return pl.kernel(
        _deg_body,
        out_type=(
            jax.ShapeDtypeStruct((NPAD,), _f32),
            jax.ShapeDtypeStruct((NPAD,), _f32),
        ),
        mesh=mesh,
        scratch_types=[
            pltpu.VMEM_SHARED((NPAD,), _f32),
            pltpu.VMEM((NCHUNK, CHUNK), jnp.int32),
            pltpu.VMEM((CHUNK,), _f32),
            pltpu.VMEM((RPT,), _f32),
        ],
    )


def _deg(dst):
    return _deg_call()(dst)


HALF = NCHUNK // 2


def _agg_body(y_hbm, zinit_hbm, src_hbm, dst_hbm, out0_hbm, out1_hbm,
              acc, src_v, dst_v, bufA, bufB, semA, semB):
    c = lax.axis_index("c")
    s = lax.axis_index("s")
    w = c * NS + s
    rows = pl.ds(s * RPT, RPT)

    pltpu.sync_copy(zinit_hbm.at[rows], acc.at[rows])

    for p in range(2):
        pltpu.sync_copy(src_hbm.at[pl.ds(w * NCHUNK + p * HALF, HALF)], src_v)
        pltpu.sync_copy(dst_hbm.at[pl.ds(w * NCHUNK + p * HALF, HALF)], dst_v)
        if p == 0:
            plsc.subcore_barrier()
        pltpu.async_copy(y_hbm.at[src_v.at[0]], bufA, semA)

        def body(j, carry):
            g = 2 * j
            pltpu.async_copy(y_hbm.at[src_v.at[g + 1]], bufB, semB)
            pltpu.make_async_copy(y_hbm.at[src_v.at[g]], bufA, semA).wait()
            pltpu.sync_copy(bufA, acc.at[dst_v.at[g]], add=True)

            @pl.when(g + 2 < HALF)
            def _():
                pltpu.async_copy(y_hbm.at[src_v.at[g + 2]], bufA, semA)

            pltpu.make_async_copy(y_hbm.at[src_v.at[g + 1]], bufB, semB).wait()
            pltpu.sync_copy(bufB, acc.at[dst_v.at[g + 1]], add=True)
            return carry

        lax.fori_loop(0, HALF // 2, body, 0)
    plsc.subcore_barrier()

    @pl.when(c == 0)
    def _():
        pltpu.sync_copy(acc.at[rows], out0_hbm.at[rows])

    @pl.when(c == 1)
    def _():
        pltpu.sync_copy(acc.at[rows], out1_hbm.at[rows])


@functools.lru_cache(maxsize=None)
def _agg_call():
    mesh = plsc.VectorSubcoreMesh(
        core_axis_name="c", subcore_axis_name="s", num_cores=NC, num_subcores=NS
    )
    return pl.kernel(
        _agg_body,
        out_type=(
            jax.ShapeDtypeStruct((NPAD, D), _f32),
            jax.ShapeDtypeStruct((NPAD, D), _f32),
        ),
        mesh=mesh,
        scratch_types=[
            pltpu.VMEM_SHARED((NPAD, D), _f32),
            pltpu.VMEM((HALF, CHUNK), jnp.int32),
            pltpu.VMEM((HALF, CHUNK), jnp.int32),
            pltpu.VMEM((CHUNK, D), _f32),
            pltpu.VMEM((CHUNK, D), _f32),
            pltpu.SemaphoreType.DMA,
            pltpu.SemaphoreType.DMA,
        ],
    )


def _agg(y, zinit, src, dst):
    return _agg_call()(y, zinit, src, dst)


_BR = 1024
_GRID = NPAD // _BR


def _mm_body(x_ref, w_ref, o_ref):
    o_ref[...] = jnp.dot(x_ref[...], w_ref[...], preferred_element_type=_f32)


_mm_call = pl.pallas_call(
    _mm_body,
    grid=(_GRID,),
    in_specs=[
        pl.BlockSpec((_BR, D), lambda i: (i, 0)),
        pl.BlockSpec((D, D), lambda i: (0, 0)),
    ],
    out_specs=pl.BlockSpec((_BR, D), lambda i: (i, 0)),
    out_shape=jax.ShapeDtypeStruct((NPAD, D), _f32),
)


def _scale_body(d0_ref, d1_ref, z_ref, y_ref, dinv_ref):
    deg = d0_ref[...] + d1_ref[...] + 1.0
    dinv = lax.rsqrt(deg)
    y_ref[...] = z_ref[...] * dinv
    dinv_ref[...] = dinv


_scale_call = pl.pallas_call(
    _scale_body,
    grid=(_GRID,),
    in_specs=[
        pl.BlockSpec((_BR, 1), lambda i: (i, 0)),
        pl.BlockSpec((_BR, 1), lambda i: (i, 0)),
        pl.BlockSpec((_BR, D), lambda i: (i, 0)),
    ],
    out_specs=[
        pl.BlockSpec((_BR, D), lambda i: (i, 0)),
        pl.BlockSpec((_BR, 1), lambda i: (i, 0)),
    ],
    out_shape=[
        jax.ShapeDtypeStruct((NPAD, D), _f32),
        jax.ShapeDtypeStruct((NPAD, 1), _f32),
    ],
)


def _layer2_body(a0_ref, a1_ref, y_ref, dinv_ref, b1_ref, w2_ref, y2_ref):
    agg = a0_ref[...] + a1_ref[...] + y_ref[...]
    h = jnp.maximum(agg * dinv_ref[...] + b1_ref[...], 0.0)
    y2 = jnp.dot(h, w2_ref[...], preferred_element_type=_f32) * dinv_ref[...]
    row = pl.program_id(0) * _BR + lax.broadcasted_iota(jnp.int32, (_BR, D), 0)
    y2_ref[...] = jnp.where(row < N, y2, 0.0)


_layer2_call = pl.pallas_call(
    _layer2_body,
    grid=(_GRID,),
    in_specs=[
        pl.BlockSpec((_BR, D), lambda i: (i, 0)),
        pl.BlockSpec((_BR, D), lambda i: (i, 0)),
        pl.BlockSpec((_BR, D), lambda i: (i, 0)),
        pl.BlockSpec((_BR, 1), lambda i: (i, 0)),
        pl.BlockSpec((1, D), lambda i: (0, 0)),
        pl.BlockSpec((D, D), lambda i: (0, 0)),
    ],
    out_specs=pl.BlockSpec((_BR, D), lambda i: (i, 0)),
    out_shape=jax.ShapeDtypeStruct((NPAD, D), _f32),
)


_BH = 1000
D_OUT = 2


def _head_body(a0_ref, a1_ref, y_ref, dinv_ref, b2_ref, wl_ref, bl_ref, o_ref):
    agg = a0_ref[...] + a1_ref[...] + y_ref[...]
    h = jnp.maximum(agg * dinv_ref[...] + b2_ref[...], 0.0)
    o_ref[...] = jnp.dot(h, wl_ref[...], preferred_element_type=_f32) + bl_ref[...]


_head_call = pl.pallas_call(
    _head_body,
    grid=(N // _BH,),
    in_specs=[
        pl.BlockSpec((_BH, D), lambda i: (i, 0)),
        pl.BlockSpec((_BH, D), lambda i: (i, 0)),
        pl.BlockSpec((_BH, D), lambda i: (i, 0)),
        pl.BlockSpec((_BH, 1), lambda i: (i, 0)),
        pl.BlockSpec((1, D), lambda i: (0, 0)),
        pl.BlockSpec((D, D_OUT), lambda i: (0, 0)),
        pl.BlockSpec((1, D_OUT), lambda i: (0, 0)),
    ],
    out_specs=pl.BlockSpec((_BH, D_OUT), lambda i: (i, 0)),
    out_shape=jax.ShapeDtypeStruct((N, D_OUT), _f32),
)


@functools.partial(jax.jit, donate_argnums=())
def kernel(x, edge_index, W1, b1, W2, b2, Wl, bl):
    ei = edge_index.astype(jnp.int32)
    pad = jnp.full((EPAD - E,), N, jnp.int32)
    src = jnp.concatenate([ei[0], pad]).reshape(IDX_ROWS, CHUNK)
    dst = jnp.concatenate([ei[1], pad]).reshape(IDX_ROWS, CHUNK)
    x_pad = jnp.concatenate([x, jnp.zeros((NPAD - N, D), _f32)], axis=0)
    zinit = jnp.zeros((NPAD, D), _f32)

    deg0, deg1 = _deg(dst)
    z1 = _mm_call(x_pad, W1)
    y1, dinv = _scale_call(deg0.reshape(NPAD, 1), deg1.reshape(NPAD, 1), z1)
    a0, a1 = _agg(y1, zinit, src, dst)
    y2 = _layer2_call(a0, a1, y1, dinv, b1.reshape(1, D), W2)
    c0, c1 = _agg(y2, zinit, src, dst)
    return _head_call(c0, c1, y2, dinv, b2.reshape(1, D), Wl, bl.reshape(1, D_OUT))


# speedup vs baseline: 3.7576x; 3.7576x over previous
import functools

import jax
import jax.numpy as jnp
from jax import lax
from jax.experimental import pallas as pl
from jax.experimental.pallas import tpu as pltpu
from jax.experimental.pallas import tpu_sc as plsc

N = 10000
D = 128
E = 320000

NC, NS, L = 2, 16, 16
NW = NC * NS
CHUNK = 128
EPT = 10240
NCHUNK = EPT // CHUNK
EPAD = EPT * NW
IDX_ROWS = EPAD // CHUNK
NPAD = 10240
RPT = NPAD // NS

_f32 = jnp.float32


def _deg_body(dst_hbm, deg0_hbm, deg1_hbm, deg_acc, dst_v, ones_v, zeros_v):
    c = lax.axis_index("c")
    s = lax.axis_index("s")
    w = c * NS + s

    def fill_zeros(i, carry):
        zeros_v[pl.ds(i * L, L)] = jnp.zeros((L,), _f32)
        return carry

    lax.fori_loop(0, RPT // L, fill_zeros, 0)

    def fill_ones(i, carry):
        ones_v[pl.ds(i * L, L)] = jnp.ones((L,), _f32)
        return carry

    lax.fori_loop(0, CHUNK // L, fill_ones, 0)

    pltpu.sync_copy(zeros_v, deg_acc.at[pl.ds(s * RPT, RPT)])
    pltpu.sync_copy(dst_hbm.at[pl.ds(w * NCHUNK, NCHUNK)], dst_v)
    plsc.subcore_barrier()

    def body(k, carry):
        pltpu.sync_copy(ones_v, deg_acc.at[dst_v.at[k]], add=True)
        return carry

    lax.fori_loop(0, NCHUNK, body, 0)
    plsc.subcore_barrier()

    rows = pl.ds(s * RPT, RPT)

    @pl.when(c == 0)
    def _():
        pltpu.sync_copy(deg_acc.at[rows], deg0_hbm.at[rows])

    @pl.when(c == 1)
    def _():
        pltpu.sync_copy(deg_acc.at[rows], deg1_hbm.at[rows])


@functools.lru_cache(maxsize=None)
def _deg_call():
    mesh = plsc.VectorSubcoreMesh(
        core_axis_name="c", subcore_axis_name="s", num_cores=NC, num_subcores=NS
    )
    return pl.kernel(
        _deg_body,
        out_type=(
            jax.ShapeDtypeStruct((NPAD,), _f32),
            jax.ShapeDtypeStruct((NPAD,), _f32),
        ),
        mesh=mesh,
        scratch_types=[
            pltpu.VMEM_SHARED((NPAD,), _f32),
            pltpu.VMEM((NCHUNK, CHUNK), jnp.int32),
            pltpu.VMEM((CHUNK,), _f32),
            pltpu.VMEM((RPT,), _f32),
        ],
    )


def _deg(dst):
    return _deg_call()(dst)


HALF = NCHUNK // 2


def _agg_body(y_hbm, zinit_hbm, src_hbm, dst_hbm, out0_hbm, out1_hbm,
              acc, src_v, dst_v, bufA, bufB, semA, semB):
    c = lax.axis_index("c")
    s = lax.axis_index("s")
    w = c * NS + s
    rows = pl.ds(s * RPT, RPT)

    pltpu.sync_copy(zinit_hbm.at[rows], acc.at[rows])

    for p in range(2):
        pltpu.sync_copy(src_hbm.at[pl.ds(w * NCHUNK + p * HALF, HALF)], src_v)
        pltpu.sync_copy(dst_hbm.at[pl.ds(w * NCHUNK + p * HALF, HALF)], dst_v)
        if p == 0:
            plsc.subcore_barrier()
        pltpu.async_copy(y_hbm.at[src_v.at[0]], bufA, semA)

        def body(j, carry):
            g = 2 * j
            pltpu.async_copy(y_hbm.at[src_v.at[g + 1]], bufB, semB)
            pltpu.make_async_copy(y_hbm.at[src_v.at[g]], bufA, semA).wait()
            pltpu.sync_copy(bufA, acc.at[dst_v.at[g]], add=True)

            @pl.when(g + 2 < HALF)
            def _():
                pltpu.async_copy(y_hbm.at[src_v.at[g + 2]], bufA, semA)

            pltpu.make_async_copy(y_hbm.at[src_v.at[g + 1]], bufB, semB).wait()
            pltpu.sync_copy(bufB, acc.at[dst_v.at[g + 1]], add=True)
            return carry

        lax.fori_loop(0, HALF // 2, body, 0)
    plsc.subcore_barrier()

    @pl.when(c == 0)
    def _():
        pltpu.sync_copy(acc.at[rows], out0_hbm.at[rows])

    @pl.when(c == 1)
    def _():
        pltpu.sync_copy(acc.at[rows], out1_hbm.at[rows])


@functools.lru_cache(maxsize=None)
def _agg_call():
    mesh = plsc.VectorSubcoreMesh(
        core_axis_name="c", subcore_axis_name="s", num_cores=NC, num_subcores=NS
    )
    return pl.kernel(
        _agg_body,
        out_type=(
            jax.ShapeDtypeStruct((NPAD, D), _f32),
            jax.ShapeDtypeStruct((NPAD, D), _f32),
        ),
        mesh=mesh,
        scratch_types=[
            pltpu.VMEM_SHARED((NPAD, D), _f32),
            pltpu.VMEM((HALF, CHUNK), jnp.int32),
            pltpu.VMEM((HALF, CHUNK), jnp.int32),
            pltpu.VMEM((CHUNK, D), _f32),
            pltpu.VMEM((CHUNK, D), _f32),
            pltpu.SemaphoreType.DMA,
            pltpu.SemaphoreType.DMA,
        ],
    )


def _agg(y, zinit, src, dst):
    return _agg_call()(y, zinit, src, dst)


_BR = 1024
_GRID = NPAD // _BR


def _mm_body(x_ref, w_ref, o_ref):
    o_ref[...] = jnp.dot(x_ref[...], w_ref[...], preferred_element_type=_f32)


_mm_call = pl.pallas_call(
    _mm_body,
    grid=(_GRID,),
    in_specs=[
        pl.BlockSpec((_BR, D), lambda i: (i, 0)),
        pl.BlockSpec((D, D), lambda i: (0, 0)),
    ],
    out_specs=pl.BlockSpec((_BR, D), lambda i: (i, 0)),
    out_shape=jax.ShapeDtypeStruct((NPAD, D), _f32),
)


def _scale_body(d0_ref, d1_ref, z_ref, y_ref, dinv_ref):
    deg = d0_ref[...] + d1_ref[...] + 1.0
    dinv = lax.rsqrt(deg)
    y_ref[...] = z_ref[...] * dinv
    dinv_ref[...] = dinv


_scale_call = pl.pallas_call(
    _scale_body,
    grid=(_GRID,),
    in_specs=[
        pl.BlockSpec((_BR, 1), lambda i: (i, 0)),
        pl.BlockSpec((_BR, 1), lambda i: (i, 0)),
        pl.BlockSpec((_BR, D), lambda i: (i, 0)),
    ],
    out_specs=[
        pl.BlockSpec((_BR, D), lambda i: (i, 0)),
        pl.BlockSpec((_BR, 1), lambda i: (i, 0)),
    ],
    out_shape=[
        jax.ShapeDtypeStruct((NPAD, D), _f32),
        jax.ShapeDtypeStruct((NPAD, 1), _f32),
    ],
)


def _layer2_body(a0_ref, a1_ref, y_ref, dinv_ref, b1_ref, w2_ref, y2_ref):
    agg = a0_ref[...] + a1_ref[...] + y_ref[...]
    h = jnp.maximum(agg * dinv_ref[...] + b1_ref[...], 0.0)
    y2 = jnp.dot(h, w2_ref[...], preferred_element_type=_f32) * dinv_ref[...]
    row = pl.program_id(0) * _BR + lax.broadcasted_iota(jnp.int32, (_BR, D), 0)
    y2_ref[...] = jnp.where(row < N, y2, 0.0)


_layer2_call = pl.pallas_call(
    _layer2_body,
    grid=(_GRID,),
    in_specs=[
        pl.BlockSpec((_BR, D), lambda i: (i, 0)),
        pl.BlockSpec((_BR, D), lambda i: (i, 0)),
        pl.BlockSpec((_BR, D), lambda i: (i, 0)),
        pl.BlockSpec((_BR, 1), lambda i: (i, 0)),
        pl.BlockSpec((1, D), lambda i: (0, 0)),
        pl.BlockSpec((D, D), lambda i: (0, 0)),
    ],
    out_specs=pl.BlockSpec((_BR, D), lambda i: (i, 0)),
    out_shape=jax.ShapeDtypeStruct((NPAD, D), _f32),
)


_BH = 1000
D_OUT = 2


def _head_body(a0_ref, a1_ref, y_ref, dinv_ref, b2_ref, wl_ref, bl_ref, o_ref):
    agg = a0_ref[...] + a1_ref[...] + y_ref[...]
    h = jnp.maximum(agg * dinv_ref[...] + b2_ref[...], 0.0)
    o_ref[...] = jnp.dot(h, wl_ref[...], preferred_element_type=_f32) + bl_ref[...]


_head_call = pl.pallas_call(
    _head_body,
    grid=(N // _BH,),
    in_specs=[
        pl.BlockSpec((_BH, D), lambda i: (i, 0)),
        pl.BlockSpec((_BH, D), lambda i: (i, 0)),
        pl.BlockSpec((_BH, D), lambda i: (i, 0)),
        pl.BlockSpec((_BH, 1), lambda i: (i, 0)),
        pl.BlockSpec((1, D), lambda i: (0, 0)),
        pl.BlockSpec((D, D_OUT), lambda i: (0, 0)),
        pl.BlockSpec((1, D_OUT), lambda i: (0, 0)),
    ],
    out_specs=pl.BlockSpec((_BH, D_OUT), lambda i: (i, 0)),
    out_shape=jax.ShapeDtypeStruct((N, D_OUT), _f32),
)


@functools.partial(jax.jit, donate_argnums=())
def kernel(x, edge_index, W1, b1, W2, b2, Wl, bl):
    ei = edge_index.astype(jnp.int32)
    pad = N + jnp.arange(EPAD - E, dtype=jnp.int32) % (NPAD - N)
    src = jnp.concatenate([ei[0], pad]).reshape(IDX_ROWS, CHUNK)
    dst = jnp.concatenate([ei[1], pad]).reshape(IDX_ROWS, CHUNK)
    x_pad = jnp.concatenate([x, jnp.zeros((NPAD - N, D), _f32)], axis=0)
    zinit = jnp.zeros((NPAD, D), _f32)

    deg0, deg1 = _deg(dst)
    z1 = _mm_call(x_pad, W1)
    y1, dinv = _scale_call(deg0.reshape(NPAD, 1), deg1.reshape(NPAD, 1), z1)
    a0, a1 = _agg(y1, zinit, src, dst)
    y2 = _layer2_call(a0, a1, y1, dinv, b1.reshape(1, D), W2)
    c0, c1 = _agg(y2, zinit, src, dst)
    return _head_call(c0, c1, y2, dinv, b2.reshape(1, D), Wl, bl.reshape(1, D_OUT))


# speedup vs baseline: 3.7614x; 1.0010x over previous
import functools

import jax
import jax.numpy as jnp
from jax import lax
from jax.experimental import pallas as pl
from jax.experimental.pallas import tpu as pltpu
from jax.experimental.pallas import tpu_sc as plsc

N = 10000
D = 128
E = 320000

NC, NS, L = 2, 16, 16
NW = NC * NS
CHUNK = 128
EPT = 10240
NCHUNK = EPT // CHUNK
EPAD = EPT * NW
IDX_ROWS = EPAD // CHUNK
NBUF = 2
NPAD = 10240
RPT = NPAD // NS

_f32 = jnp.float32


def _deg_body(dst_hbm, deg0_hbm, deg1_hbm, deg_acc, dst_v, ones_v, zeros_v):
    c = lax.axis_index("c")
    s = lax.axis_index("s")
    w = c * NS + s

    def fill_zeros(i, carry):
        zeros_v[pl.ds(i * L, L)] = jnp.zeros((L,), _f32)
        return carry

    lax.fori_loop(0, RPT // L, fill_zeros, 0)

    def fill_ones(i, carry):
        ones_v[pl.ds(i * L, L)] = jnp.ones((L,), _f32)
        return carry

    lax.fori_loop(0, CHUNK // L, fill_ones, 0)

    pltpu.sync_copy(zeros_v, deg_acc.at[pl.ds(s * RPT, RPT)])
    pltpu.sync_copy(dst_hbm.at[pl.ds(w * NCHUNK, NCHUNK)], dst_v)
    plsc.subcore_barrier()

    def body(k, carry):
        pltpu.sync_copy(ones_v, deg_acc.at[dst_v.at[k]], add=True)
        return carry

    lax.fori_loop(0, NCHUNK, body, 0)
    plsc.subcore_barrier()

    rows = pl.ds(s * RPT, RPT)

    @pl.when(c == 0)
    def _():
        pltpu.sync_copy(deg_acc.at[rows], deg0_hbm.at[rows])

    @pl.when(c == 1)
    def _():
        pltpu.sync_copy(deg_acc.at[rows], deg1_hbm.at[rows])


@functools.lru_cache(maxsize=None)
def _deg_call():
    mesh = plsc.VectorSubcoreMesh(
        core_axis_name="c", subcore_axis_name="s", num_cores=NC, num_subcores=NS
    )
    return pl.kernel(
        _deg_body,
        out_type=(
            jax.ShapeDtypeStruct((NPAD,), _f32),
            jax.ShapeDtypeStruct((NPAD,), _f32),
        ),
        mesh=mesh,
        scratch_types=[
            pltpu.VMEM_SHARED((NPAD,), _f32),
            pltpu.VMEM((NCHUNK, CHUNK), jnp.int32),
            pltpu.VMEM((CHUNK,), _f32),
            pltpu.VMEM((RPT,), _f32),
        ],
    )


def _deg(dst):
    return _deg_call()(dst)


HALF = NCHUNK // 2


def _agg_body(y_hbm, zinit_hbm, src_hbm, dst_hbm, out0_hbm, out1_hbm,
              acc, src_v, dst_v, buf0, buf1, sem0, sem1):
    c = lax.axis_index("c")
    s = lax.axis_index("s")
    w = c * NS + s
    rows = pl.ds(s * RPT, RPT)

    pltpu.sync_copy(zinit_hbm, acc.at[rows])

    bufs = (buf0, buf1)
    sems = (sem0, sem1)

    for p in range(2):
        pltpu.sync_copy(src_hbm.at[pl.ds(w * NCHUNK + p * HALF, HALF)], src_v)
        pltpu.sync_copy(dst_hbm.at[pl.ds(w * NCHUNK + p * HALF, HALF)], dst_v)
        if p == 0:
            plsc.subcore_barrier()
        for i in range(NBUF):
            pltpu.async_copy(y_hbm.at[src_v.at[i]], bufs[i], sems[i])

        def body(j, carry):
            g = NBUF * j
            for i in range(NBUF):
                pltpu.make_async_copy(y_hbm.at[src_v.at[g + i]], bufs[i], sems[i]).wait()
                pltpu.sync_copy(bufs[i], acc.at[dst_v.at[g + i]], add=True)

                @pl.when(g + NBUF + i < HALF)
                def _():
                    pltpu.async_copy(y_hbm.at[src_v.at[g + NBUF + i]], bufs[i], sems[i])
            return carry

        lax.fori_loop(0, HALF // NBUF, body, 0)
    plsc.subcore_barrier()

    @pl.when(c == 0)
    def _():
        pltpu.sync_copy(acc.at[rows], out0_hbm.at[rows])

    @pl.when(c == 1)
    def _():
        pltpu.sync_copy(acc.at[rows], out1_hbm.at[rows])


@functools.lru_cache(maxsize=None)
def _agg_call():
    mesh = plsc.VectorSubcoreMesh(
        core_axis_name="c", subcore_axis_name="s", num_cores=NC, num_subcores=NS
    )
    return pl.kernel(
        _agg_body,
        out_type=(
            jax.ShapeDtypeStruct((NPAD, D), _f32),
            jax.ShapeDtypeStruct((NPAD, D), _f32),
        ),
        mesh=mesh,
        scratch_types=[
            pltpu.VMEM_SHARED((NPAD, D), _f32),
            pltpu.VMEM((HALF, CHUNK), jnp.int32),
            pltpu.VMEM((HALF, CHUNK), jnp.int32),
            pltpu.VMEM((CHUNK, D), _f32),
            pltpu.VMEM((CHUNK, D), _f32),
            pltpu.SemaphoreType.DMA,
            pltpu.SemaphoreType.DMA,
        ],
    )


def _agg(y, zinit, src, dst):
    return _agg_call()(y, zinit, src, dst)


_BR = 1024
_GRID = NPAD // _BR


def _mm_body(x_ref, w_ref, o_ref):
    o_ref[...] = jnp.dot(x_ref[...], w_ref[...], preferred_element_type=_f32)


_mm_call = pl.pallas_call(
    _mm_body,
    grid=(_GRID,),
    in_specs=[
        pl.BlockSpec((_BR, D), lambda i: (i, 0)),
        pl.BlockSpec((D, D), lambda i: (0, 0)),
    ],
    out_specs=pl.BlockSpec((_BR, D), lambda i: (i, 0)),
    out_shape=jax.ShapeDtypeStruct((NPAD, D), _f32),
)


def _scale_body(d0_ref, d1_ref, z_ref, y_ref, dinv_ref):
    deg = d0_ref[...] + d1_ref[...] + 1.0
    dinv = lax.rsqrt(deg)
    y_ref[...] = z_ref[...] * dinv
    dinv_ref[...] = dinv


_scale_call = pl.pallas_call(
    _scale_body,
    grid=(_GRID,),
    in_specs=[
        pl.BlockSpec((_BR, 1), lambda i: (i, 0)),
        pl.BlockSpec((_BR, 1), lambda i: (i, 0)),
        pl.BlockSpec((_BR, D), lambda i: (i, 0)),
    ],
    out_specs=[
        pl.BlockSpec((_BR, D), lambda i: (i, 0)),
        pl.BlockSpec((_BR, 1), lambda i: (i, 0)),
    ],
    out_shape=[
        jax.ShapeDtypeStruct((NPAD, D), _f32),
        jax.ShapeDtypeStruct((NPAD, 1), _f32),
    ],
)


def _layer2_body(a0_ref, a1_ref, y_ref, dinv_ref, b1_ref, w2_ref, y2_ref):
    agg = a0_ref[...] + a1_ref[...] + y_ref[...]
    h = jnp.maximum(agg * dinv_ref[...] + b1_ref[...], 0.0)
    y2 = jnp.dot(h, w2_ref[...], preferred_element_type=_f32) * dinv_ref[...]
    row = pl.program_id(0) * _BR + lax.broadcasted_iota(jnp.int32, (_BR, D), 0)
    y2_ref[...] = jnp.where(row < N, y2, 0.0)


_layer2_call = pl.pallas_call(
    _layer2_body,
    grid=(_GRID,),
    in_specs=[
        pl.BlockSpec((_BR, D), lambda i: (i, 0)),
        pl.BlockSpec((_BR, D), lambda i: (i, 0)),
        pl.BlockSpec((_BR, D), lambda i: (i, 0)),
        pl.BlockSpec((_BR, 1), lambda i: (i, 0)),
        pl.BlockSpec((1, D), lambda i: (0, 0)),
        pl.BlockSpec((D, D), lambda i: (0, 0)),
    ],
    out_specs=pl.BlockSpec((_BR, D), lambda i: (i, 0)),
    out_shape=jax.ShapeDtypeStruct((NPAD, D), _f32),
)


_BH = 1000
D_OUT = 2


def _head_body(a0_ref, a1_ref, y_ref, dinv_ref, b2_ref, wl_ref, bl_ref, o_ref):
    agg = a0_ref[...] + a1_ref[...] + y_ref[...]
    h = jnp.maximum(agg * dinv_ref[...] + b2_ref[...], 0.0)
    o_ref[...] = jnp.dot(h, wl_ref[...], preferred_element_type=_f32) + bl_ref[...]


_head_call = pl.pallas_call(
    _head_body,
    grid=(N // _BH,),
    in_specs=[
        pl.BlockSpec((_BH, D), lambda i: (i, 0)),
        pl.BlockSpec((_BH, D), lambda i: (i, 0)),
        pl.BlockSpec((_BH, D), lambda i: (i, 0)),
        pl.BlockSpec((_BH, 1), lambda i: (i, 0)),
        pl.BlockSpec((1, D), lambda i: (0, 0)),
        pl.BlockSpec((D, D_OUT), lambda i: (0, 0)),
        pl.BlockSpec((1, D_OUT), lambda i: (0, 0)),
    ],
    out_specs=pl.BlockSpec((_BH, D_OUT), lambda i: (i, 0)),
    out_shape=jax.ShapeDtypeStruct((N, D_OUT), _f32),
)


@functools.partial(jax.jit, donate_argnums=())
def kernel(x, edge_index, W1, b1, W2, b2, Wl, bl):
    ei = edge_index.astype(jnp.int32)
    pad = N + jnp.arange(EPAD - E, dtype=jnp.int32) % (NPAD - N)
    src = jnp.concatenate([ei[0], pad]).reshape(IDX_ROWS, CHUNK)
    dst = jnp.concatenate([ei[1], pad]).reshape(IDX_ROWS, CHUNK)
    x_pad = jnp.concatenate([x, jnp.zeros((NPAD - N, D), _f32)], axis=0)
    zinit = jnp.zeros((RPT, D), _f32)

    deg0, deg1 = _deg(dst)
    z1 = _mm_call(x_pad, W1)
    y1, dinv = _scale_call(deg0.reshape(NPAD, 1), deg1.reshape(NPAD, 1), z1)
    a0, a1 = _agg(y1, zinit, src, dst)
    y2 = _layer2_call(a0, a1, y1, dinv, b1.reshape(1, D), W2)
    c0, c1 = _agg(y2, zinit, src, dst)
    return _head_call(c0, c1, y2, dinv, b2.reshape(1, D), Wl, bl.reshape(1, D_OUT))


# speedup vs baseline: 3.7913x; 1.0079x over previous
import functools

import jax
import jax.numpy as jnp
from jax import lax
from jax.experimental import pallas as pl
from jax.experimental.pallas import tpu as pltpu
from jax.experimental.pallas import tpu_sc as plsc

N = 10000
D = 128
E = 320000

NC, NS, L = 2, 16, 16
NW = NC * NS
CHUNK = 128
EPT = 10240
NCHUNK = EPT // CHUNK
EPAD = EPT * NW
IDX_ROWS = EPAD // CHUNK
NBUF = 2
NPAD = 10240
RPT = NPAD // NS

_f32 = jnp.float32


def _deg_body(dst_hbm, deg0_hbm, deg1_hbm, deg_acc, dst_v, ones_v, zeros_v):
    c = lax.axis_index("c")
    s = lax.axis_index("s")
    w = c * NS + s

    def fill_zeros(i, carry):
        zeros_v[pl.ds(i * L, L)] = jnp.zeros((L,), _f32)
        return carry

    lax.fori_loop(0, RPT // L, fill_zeros, 0)

    def fill_ones(i, carry):
        ones_v[pl.ds(i * L, L)] = jnp.ones((L,), _f32)
        return carry

    lax.fori_loop(0, CHUNK // L, fill_ones, 0)

    pltpu.sync_copy(zeros_v, deg_acc.at[pl.ds(s * RPT, RPT)])
    pltpu.sync_copy(dst_hbm.at[pl.ds(w * NCHUNK, NCHUNK)], dst_v)
    plsc.subcore_barrier()

    def body(k, carry):
        pltpu.sync_copy(ones_v, deg_acc.at[dst_v.at[k]], add=True)
        return carry

    lax.fori_loop(0, NCHUNK, body, 0)
    plsc.subcore_barrier()

    rows = pl.ds(s * RPT, RPT)

    @pl.when(c == 0)
    def _():
        pltpu.sync_copy(deg_acc.at[rows], deg0_hbm.at[rows])

    @pl.when(c == 1)
    def _():
        pltpu.sync_copy(deg_acc.at[rows], deg1_hbm.at[rows])


@functools.lru_cache(maxsize=None)
def _deg_call():
    mesh = plsc.VectorSubcoreMesh(
        core_axis_name="c", subcore_axis_name="s", num_cores=NC, num_subcores=NS
    )
    return pl.kernel(
        _deg_body,
        out_type=(
            jax.ShapeDtypeStruct((NPAD,), _f32),
            jax.ShapeDtypeStruct((NPAD,), _f32),
        ),
        mesh=mesh,
        scratch_types=[
            pltpu.VMEM_SHARED((NPAD,), _f32),
            pltpu.VMEM((NCHUNK, CHUNK), jnp.int32),
            pltpu.VMEM((CHUNK,), _f32),
            pltpu.VMEM((RPT,), _f32),
        ],
    )


def _deg(dst):
    return _deg_call()(dst)


HALF = NCHUNK // 2


def _agg_body(y_hbm, src_hbm, dst_hbm, out0_hbm, out1_hbm,
              acc, src_v, dst_v, buf0, buf1, sem0, sem1):
    c = lax.axis_index("c")
    s = lax.axis_index("s")
    w = c * NS + s
    rows = pl.ds(s * RPT, RPT)

    pltpu.sync_copy(y_hbm.at[rows], acc.at[rows])

    bufs = (buf0, buf1)
    sems = (sem0, sem1)

    for p in range(2):
        pltpu.sync_copy(src_hbm.at[pl.ds(w * NCHUNK + p * HALF, HALF)], src_v)
        pltpu.sync_copy(dst_hbm.at[pl.ds(w * NCHUNK + p * HALF, HALF)], dst_v)
        if p == 0:
            plsc.subcore_barrier()
        for i in range(NBUF):
            pltpu.async_copy(y_hbm.at[src_v.at[i]], bufs[i], sems[i])

        def body(j, carry):
            g = NBUF * j
            for i in range(NBUF):
                pltpu.make_async_copy(y_hbm.at[src_v.at[g + i]], bufs[i], sems[i]).wait()
                pltpu.sync_copy(bufs[i], acc.at[dst_v.at[g + i]], add=True)

                @pl.when(g + NBUF + i < HALF)
                def _():
                    pltpu.async_copy(y_hbm.at[src_v.at[g + NBUF + i]], bufs[i], sems[i])
            return carry

        lax.fori_loop(0, HALF // NBUF, body, 0)
    plsc.subcore_barrier()

    @pl.when(c == 0)
    def _():
        pltpu.sync_copy(acc.at[rows], out0_hbm.at[rows])

    @pl.when(c == 1)
    def _():
        pltpu.sync_copy(acc.at[rows], out1_hbm.at[rows])


@functools.lru_cache(maxsize=None)
def _agg_call():
    mesh = plsc.VectorSubcoreMesh(
        core_axis_name="c", subcore_axis_name="s", num_cores=NC, num_subcores=NS
    )
    return pl.kernel(
        _agg_body,
        out_type=(
            jax.ShapeDtypeStruct((NPAD, D), _f32),
            jax.ShapeDtypeStruct((NPAD, D), _f32),
        ),
        mesh=mesh,
        scratch_types=[
            pltpu.VMEM_SHARED((NPAD, D), _f32),
            pltpu.VMEM((HALF, CHUNK), jnp.int32),
            pltpu.VMEM((HALF, CHUNK), jnp.int32),
            pltpu.VMEM((CHUNK, D), _f32),
            pltpu.VMEM((CHUNK, D), _f32),
            pltpu.SemaphoreType.DMA,
            pltpu.SemaphoreType.DMA,
        ],
    )


def _agg(y, src, dst):
    return _agg_call()(y, src, dst)


_BR = 1024
_GRID = NPAD // _BR


def _mm_body(x_ref, w_ref, o_ref):
    o_ref[...] = jnp.dot(x_ref[...], w_ref[...], preferred_element_type=_f32)


_mm_call = pl.pallas_call(
    _mm_body,
    grid=(_GRID,),
    in_specs=[
        pl.BlockSpec((_BR, D), lambda i: (i, 0)),
        pl.BlockSpec((D, D), lambda i: (0, 0)),
    ],
    out_specs=pl.BlockSpec((_BR, D), lambda i: (i, 0)),
    out_shape=jax.ShapeDtypeStruct((NPAD, D), _f32),
)


def _scale_body(d0_ref, d1_ref, z_ref, y_ref, dinv_ref):
    deg = d0_ref[...] + d1_ref[...] + 1.0
    dinv = lax.rsqrt(deg)
    y_ref[...] = z_ref[...] * dinv
    dinv_ref[...] = dinv


_scale_call = pl.pallas_call(
    _scale_body,
    grid=(_GRID,),
    in_specs=[
        pl.BlockSpec((_BR, 1), lambda i: (i, 0)),
        pl.BlockSpec((_BR, 1), lambda i: (i, 0)),
        pl.BlockSpec((_BR, D), lambda i: (i, 0)),
    ],
    out_specs=[
        pl.BlockSpec((_BR, D), lambda i: (i, 0)),
        pl.BlockSpec((_BR, 1), lambda i: (i, 0)),
    ],
    out_shape=[
        jax.ShapeDtypeStruct((NPAD, D), _f32),
        jax.ShapeDtypeStruct((NPAD, 1), _f32),
    ],
)


def _layer2_body(a0_ref, a1_ref, y_ref, dinv_ref, b1_ref, w2_ref, y2_ref):
    agg = a0_ref[...] + a1_ref[...] - y_ref[...]
    h = jnp.maximum(agg * dinv_ref[...] + b1_ref[...], 0.0)
    y2 = jnp.dot(h, w2_ref[...], preferred_element_type=_f32) * dinv_ref[...]
    row = pl.program_id(0) * _BR + lax.broadcasted_iota(jnp.int32, (_BR, D), 0)
    y2_ref[...] = jnp.where(row < N, y2, 0.0)


_layer2_call = pl.pallas_call(
    _layer2_body,
    grid=(_GRID,),
    in_specs=[
        pl.BlockSpec((_BR, D), lambda i: (i, 0)),
        pl.BlockSpec((_BR, D), lambda i: (i, 0)),
        pl.BlockSpec((_BR, D), lambda i: (i, 0)),
        pl.BlockSpec((_BR, 1), lambda i: (i, 0)),
        pl.BlockSpec((1, D), lambda i: (0, 0)),
        pl.BlockSpec((D, D), lambda i: (0, 0)),
    ],
    out_specs=pl.BlockSpec((_BR, D), lambda i: (i, 0)),
    out_shape=jax.ShapeDtypeStruct((NPAD, D), _f32),
)


_BH = 1000
D_OUT = 2


def _head_body(a0_ref, a1_ref, y_ref, dinv_ref, b2_ref, wl_ref, bl_ref, o_ref):
    agg = a0_ref[...] + a1_ref[...] - y_ref[...]
    h = jnp.maximum(agg * dinv_ref[...] + b2_ref[...], 0.0)
    o_ref[...] = jnp.dot(h, wl_ref[...], preferred_element_type=_f32) + bl_ref[...]


_head_call = pl.pallas_call(
    _head_body,
    grid=(N // _BH,),
    in_specs=[
        pl.BlockSpec((_BH, D), lambda i: (i, 0)),
        pl.BlockSpec((_BH, D), lambda i: (i, 0)),
        pl.BlockSpec((_BH, D), lambda i: (i, 0)),
        pl.BlockSpec((_BH, 1), lambda i: (i, 0)),
        pl.BlockSpec((1, D), lambda i: (0, 0)),
        pl.BlockSpec((D, D_OUT), lambda i: (0, 0)),
        pl.BlockSpec((1, D_OUT), lambda i: (0, 0)),
    ],
    out_specs=pl.BlockSpec((_BH, D_OUT), lambda i: (i, 0)),
    out_shape=jax.ShapeDtypeStruct((N, D_OUT), _f32),
)


@functools.partial(jax.jit, donate_argnums=())
def kernel(x, edge_index, W1, b1, W2, b2, Wl, bl):
    ei = edge_index.astype(jnp.int32)
    pad = N + jnp.arange(EPAD - E, dtype=jnp.int32) % (NPAD - N)
    src = jnp.concatenate([ei[0], pad]).reshape(IDX_ROWS, CHUNK)
    dst = jnp.concatenate([ei[1], pad]).reshape(IDX_ROWS, CHUNK)
    x_pad = jnp.concatenate([x, jnp.zeros((NPAD - N, D), _f32)], axis=0)

    deg0, deg1 = _deg(dst)
    z1 = _mm_call(x_pad, W1)
    y1, dinv = _scale_call(deg0.reshape(NPAD, 1), deg1.reshape(NPAD, 1), z1)
    a0, a1 = _agg(y1, src, dst)
    y2 = _layer2_call(a0, a1, y1, dinv, b1.reshape(1, D), W2)
    c0, c1 = _agg(y2, src, dst)
    return _head_call(c0, c1, y2, dinv, b2.reshape(1, D), Wl, bl.reshape(1, D_OUT))
